# baseline (speedup 1.0000x reference)
import functools

import jax
import jax.numpy as jnp
from jax import lax
from jax.experimental import pallas as pl
from jax.experimental.pallas import tpu as pltpu
from jax.experimental.pallas import tpu_sc as plsc

ENT_DIM = 128
REL_DIM = 64
GAMMA = 12.0
NC = 2
NS = 16
NW = NC * NS
LANES = 16
HALF = ENT_DIM * REL_DIM // 2


@functools.lru_cache(maxsize=None)
def _make_sc_call(batch):
    SPW = batch // NW
    CH = 4
    NCH = SPW // CH
    KC = ENT_DIM // LANES
    JC = REL_DIM // LANES

    mesh = plsc.VectorSubcoreMesh(core_axis_name="c", subcore_axis_name="s")

    @functools.partial(
        pl.kernel,
        mesh=mesh,
        compiler_params=pltpu.CompilerParams(needs_layout_passes=False),
        out_type=jax.ShapeDtypeStruct((batch,), jnp.float32),
        scratch_types=[
            pltpu.VMEM((SPW,), jnp.int32),
            pltpu.VMEM((SPW,), jnp.int32),
            pltpu.VMEM((SPW,), jnp.int32),
            pltpu.VMEM((2 * SPW,), jnp.int32),
            pltpu.VMEM((SPW, ENT_DIM), jnp.float32),
            pltpu.VMEM((SPW, ENT_DIM), jnp.float32),
            pltpu.VMEM((SPW, ENT_DIM), jnp.float32),
            pltpu.VMEM((2 * CH, HALF), jnp.float32),
            pltpu.VMEM((SPW * LANES,), jnp.float32),
            pltpu.VMEM((SPW,), jnp.float32),
            pltpu.SemaphoreType.DMA,
            pltpu.SemaphoreType.DMA,
            pltpu.SemaphoreType.DMA,
            pltpu.SemaphoreType.DMA,
        ],
    )
    def call(hidx_hbm, tidx_hbm, ridx_hbm, midx_hbm, ent_hbm, rel_hbm,
             mr2_hbm, out_hbm, hidx_v, tidx_v, ridx_v, midx_v, d_rows,
             t_rows, rel_rows, mbuf, nsqf, scr_v, sem_h, sem_t, sem_r,
             sem_m):
        wid = lax.axis_index("s") * NC + lax.axis_index("c")
        base = wid * SPW

        pltpu.sync_copy(hidx_hbm.at[pl.ds(base, SPW)], hidx_v)
        pltpu.sync_copy(tidx_hbm.at[pl.ds(base, SPW)], tidx_v)
        pltpu.sync_copy(ridx_hbm.at[pl.ds(base, SPW)], ridx_v)
        pltpu.sync_copy(midx_hbm.at[pl.ds(2 * base, 2 * SPW)], midx_v)

        cp_h = pltpu.async_copy(ent_hbm.at[hidx_v], d_rows, sem_h)
        cp_t = pltpu.async_copy(ent_hbm.at[tidx_v], t_rows, sem_t)
        cp_r = pltpu.async_copy(rel_hbm.at[ridx_v], rel_rows, sem_r)
        cp_h.wait()
        cp_t.wait()
        cp_r.wait()

        def dbody(s, carry):
            for c in range(KC):
                sl = pl.ds(c * LANES, LANES)
                d_rows[s, sl] = d_rows[s, sl] - t_rows[s, sl]
            return carry

        lax.fori_loop(0, SPW, dbody, 0)

        iota = lax.iota(jnp.int32, LANES)
        zero = jnp.zeros((LANES,), jnp.float32)
        col_base = [iota * ENT_DIM, iota * ENT_DIM + 32 * ENT_DIM // 2,
                    iota * ENT_DIM, iota * ENT_DIM + 32 * ENT_DIM // 2]

        def chunk_body(c, carry):
            pltpu.async_copy(
                mr2_hbm.at[midx_v.at[pl.ds(c * 2 * CH, 2 * CH)]], mbuf, sem_m
            ).wait()

            def sample_body(sloc, carry2):
                s = c * CH + sloc
                svec = jnp.full((LANES,), s, jnp.int32)
                row_a = jnp.full((LANES,), 2 * sloc, jnp.int32)
                row_b = row_a + 1
                rows = [row_a, row_a, row_b, row_b]

                def kc_body(kc, accs):
                    accs = list(accs)
                    kb = kc * LANES
                    for lane in range(LANES):
                        kk = kb + lane
                        kvec = jnp.full((LANES,), kk, jnp.int32)
                        dbk = plsc.load_gather(d_rows, [svec, kvec])
                        for jc in range(JC):
                            mv = plsc.load_gather(
                                mbuf, [rows[jc], col_base[jc] + kk])
                            accs[jc] = accs[jc] + mv * dbk
                    return tuple(accs)

                accs = lax.fori_loop(0, KC, kc_body, (zero,) * JC)

                nv = zero
                for jc in range(JC):
                    sc = accs[jc] + rel_rows[s, pl.ds(jc * LANES, LANES)]
                    nv = nv + sc * sc
                nsqf[pl.ds(s * LANES, LANES)] = nv
                return carry2

            lax.fori_loop(0, CH, sample_body, 0)
            return carry

        lax.fori_loop(0, NCH, chunk_body, 0)

        def sq_body(g, carry):
            bvec = g * (LANES * LANES) + iota * LANES
            v = zero
            for lane in range(LANES):
                v = v + plsc.load_gather(nsqf, [bvec + lane])
            yi = lax.bitcast_convert_type(v, jnp.int32)
            yi = jnp.int32(0x5F3759DF) - lax.shift_right_logical(yi, 1)
            y = lax.bitcast_convert_type(yi, jnp.float32)
            for _ in range(3):
                y = y * (1.5 - 0.5 * v * y * y)
            scr_v[pl.ds(g * LANES, LANES)] = v * y - GAMMA
            return carry

        lax.fori_loop(0, SPW // LANES, sq_body, 0)

        pltpu.sync_copy(scr_v, out_hbm.at[pl.ds(base, SPW)])

    return call


def kernel(pos_sample, ent_embd, rel_embd, mr):
    batch = pos_sample.shape[0]
    idx = pos_sample.astype(jnp.int32)
    h_idx = idx[:, 0]
    r_idx = idx[:, 1]
    t_idx = idx[:, 2]
    midx = jnp.stack([2 * r_idx, 2 * r_idx + 1], axis=-1).reshape(-1)
    mr2 = mr.reshape(-1, HALF)
    rel_pad = jnp.pad(rel_embd, ((0, 0), (0, ENT_DIM - REL_DIM)))
    call = _make_sc_call(batch)
    scores = call(h_idx, t_idx, r_idx, midx, ent_embd, rel_pad, mr2)
    return scores.reshape(batch, 1)


# speedup vs baseline: 5.7200x; 5.7200x over previous
import functools

import jax
import jax.numpy as jnp
from jax import lax
from jax.experimental import pallas as pl
from jax.experimental.pallas import tpu as pltpu
from jax.experimental.pallas import tpu_sc as plsc

ENT_DIM = 128
REL_DIM = 64
GAMMA = 12.0
NC = 2
NS = 16
NW = NC * NS
LANES = 16
HALF = ENT_DIM * REL_DIM // 2


@functools.lru_cache(maxsize=None)
def _make_sc_call(batch):
    SPW = batch // NW
    CH = 4
    NCH = SPW // CH
    KC = ENT_DIM // LANES
    JC = REL_DIM // LANES

    mesh = plsc.VectorSubcoreMesh(core_axis_name="c", subcore_axis_name="s")

    @functools.partial(
        pl.kernel,
        mesh=mesh,
        compiler_params=pltpu.CompilerParams(needs_layout_passes=False),
        out_type=jax.ShapeDtypeStruct((batch,), jnp.float32),
        scratch_types=[
            pltpu.VMEM((SPW,), jnp.int32),
            pltpu.VMEM((SPW,), jnp.int32),
            pltpu.VMEM((SPW,), jnp.int32),
            pltpu.VMEM((2 * SPW,), jnp.int32),
            pltpu.VMEM((SPW, ENT_DIM), jnp.float32),
            pltpu.VMEM((SPW, ENT_DIM), jnp.float32),
            pltpu.VMEM((SPW, ENT_DIM), jnp.float32),
            pltpu.VMEM((2 * CH, HALF), jnp.float32),
            pltpu.VMEM((2 * CH, HALF), jnp.float32),
            pltpu.VMEM((SPW,), jnp.float32),
            pltpu.VMEM((SPW,), jnp.float32),
            pltpu.SemaphoreType.DMA,
            pltpu.SemaphoreType.DMA,
            pltpu.SemaphoreType.DMA,
            pltpu.SemaphoreType.DMA,
            pltpu.SemaphoreType.DMA,
        ],
    )
    def call(hidx_hbm, tidx_hbm, ridx_hbm, midx_hbm, ent_hbm, rel_hbm,
             mr2_hbm, out_hbm, hidx_v, tidx_v, ridx_v, midx_v, d_rows,
             t_rows, rel_rows, mbuf0, mbuf1, nsq_v, scr_v, sem_h, sem_t,
             sem_r, sem_m0, sem_m1):
        wid = lax.axis_index("s") * NC + lax.axis_index("c")
        base = wid * SPW

        pltpu.sync_copy(hidx_hbm.at[pl.ds(base, SPW)], hidx_v)
        pltpu.sync_copy(tidx_hbm.at[pl.ds(base, SPW)], tidx_v)
        pltpu.sync_copy(ridx_hbm.at[pl.ds(base, SPW)], ridx_v)
        pltpu.sync_copy(midx_hbm.at[pl.ds(2 * base, 2 * SPW)], midx_v)

        def mcopy(c, buf, sem):
            return pltpu.make_async_copy(
                mr2_hbm.at[midx_v.at[pl.ds(c * 2 * CH, 2 * CH)]], buf, sem)

        mcopy(0, mbuf0, sem_m0).start()
        mcopy(1, mbuf1, sem_m1).start()

        cp_h = pltpu.async_copy(ent_hbm.at[hidx_v], d_rows, sem_h)
        cp_t = pltpu.async_copy(ent_hbm.at[tidx_v], t_rows, sem_t)
        cp_r = pltpu.async_copy(rel_hbm.at[ridx_v], rel_rows, sem_r)
        cp_h.wait()
        cp_t.wait()
        cp_r.wait()

        def dbody(s, carry):
            for c in range(KC):
                sl = pl.ds(c * LANES, LANES)
                d_rows[s, sl] = d_rows[s, sl] - t_rows[s, sl]
            return carry

        lax.fori_loop(0, SPW, dbody, 0)

        iota = lax.iota(jnp.int32, LANES)
        lane0 = iota == 0

        def sample_body(sloc, c, buf):
            s = c * CH + sloc
            dch = [d_rows[s, pl.ds(kc * LANES, LANES)] for kc in range(KC)]
            relch = [rel_rows[s, pl.ds(jc * LANES, LANES)] for jc in range(JC)]
            nsq = jnp.float32(0.0)
            for jc in range(JC):
                row = 2 * sloc + (jc // 2)
                cbase = (jc % 2) * (16 * ENT_DIM)
                for j16 in range(LANES):
                    colb = cbase + j16 * ENT_DIM
                    p = [buf[row, pl.ds(colb + kc * LANES, LANES)] * dch[kc]
                         for kc in range(KC)]
                    acc = ((p[0] + p[1]) + (p[2] + p[3])) + (
                        (p[4] + p[5]) + (p[6] + p[7]))
                    sj = jnp.sum(acc) + relch[jc][j16]
                    nsq = nsq + sj * sj
            plsc.store_scatter(nsq_v, [jnp.full((LANES,), s, jnp.int32)],
                               jnp.full((LANES,), nsq, jnp.float32),
                               mask=lane0)

        def super_body(g, carry):
            for b, (buf, sem) in enumerate(((mbuf0, sem_m0), (mbuf1, sem_m1))):
                c = 2 * g + b
                mcopy(c, buf, sem).wait()

                def sb(sloc, carry2):
                    sample_body(sloc, c, buf)
                    return carry2

                lax.fori_loop(0, CH, sb, 0)

                @pl.when(c + 2 < NCH)
                def _():
                    mcopy(c + 2, buf, sem).start()
            return carry

        lax.fori_loop(0, NCH // 2, super_body, 0)

        def sq_body(i, carry):
            sl = pl.ds(i * LANES, LANES)
            v = nsq_v[sl]
            yi = lax.bitcast_convert_type(v, jnp.int32)
            yi = jnp.int32(0x5F3759DF) - lax.shift_right_logical(yi, 1)
            y = lax.bitcast_convert_type(yi, jnp.float32)
            for _ in range(3):
                y = y * (1.5 - 0.5 * v * y * y)
            scr_v[sl] = v * y - GAMMA
            return carry

        lax.fori_loop(0, SPW // LANES, sq_body, 0)

        pltpu.sync_copy(scr_v, out_hbm.at[pl.ds(base, SPW)])

    return call


def kernel(pos_sample, ent_embd, rel_embd, mr):
    batch = pos_sample.shape[0]
    idx = pos_sample.astype(jnp.int32)
    h_idx = idx[:, 0]
    r_idx = idx[:, 1]
    t_idx = idx[:, 2]
    midx = jnp.stack([2 * r_idx, 2 * r_idx + 1], axis=-1).reshape(-1)
    mr2 = mr.reshape(-1, HALF)
    rel_pad = jnp.pad(rel_embd, ((0, 0), (0, ENT_DIM - REL_DIM)))
    call = _make_sc_call(batch)
    scores = call(h_idx, t_idx, r_idx, midx, ent_embd, rel_pad, mr2)
    return scores.reshape(batch, 1)
